# baseline (speedup 1.0000x reference)
import functools

import jax
import jax.numpy as jnp
from jax import lax
from jax.experimental import pallas as pl
from jax.experimental.pallas import tpu as pltpu
from jax.experimental.pallas import tpu_sc as plsc

_NC = 2
_NS = 16
_NW = _NC * _NS
_G = 7


def _make_gather(b, f, d):
  n = b * _G * 4
  per_w = n // _NW
  spw = b // _NW
  mesh = plsc.VectorSubcoreMesh(core_axis_name="c", subcore_axis_name="s")

  @functools.partial(
      pl.kernel,
      out_type=(
          jax.ShapeDtypeStruct((n, d), jnp.float32),
          jax.ShapeDtypeStruct((b, 128), jnp.float32),
      ),
      mesh=mesh,
      compiler_params=pltpu.CompilerParams(use_tc_tiling_on_sc=False,
                                           needs_layout_passes=False),
      scratch_types=[
          pltpu.VMEM((per_w,), jnp.int32),
          pltpu.VMEM((per_w, d), jnp.float32),
          pltpu.VMEM((per_w,), jnp.float32),
          pltpu.VMEM((spw, 1), jnp.float32),
          pltpu.SemaphoreType.DMA,
          pltpu.SemaphoreType.DMA,
      ],
  )
  def gather(idx_hbm, emb_hbm, wide_hbm, emb_out, wsum_out,
             idx_v, rows_v, wvals_v, wsum_v, sem, wsem):
    wid = lax.axis_index("s") * _NC + lax.axis_index("c")
    base = wid * per_w
    pltpu.sync_copy(idx_hbm.at[pl.ds(base, per_w)], idx_v)
    cp = pltpu.async_copy(emb_hbm.at[idx_v], rows_v, sem)
    wp = pltpu.async_copy(wide_hbm.at[idx_v], wvals_v, wsem)
    cp.wait()
    wp.wait()
    lane = lax.iota(jnp.int32, 16)
    zero16 = jnp.zeros((16,), jnp.int32)

    def _one(g, _):
      t16 = g * 16 + lane
      boff = (t16 >> 3) * (32 * _G) + (t16 & 7) * 4

      def _k(k, acc):
        off = boff + (k >> 2) * 32 + (k & 3)
        return acc + plsc.load_gather(wvals_v, [off])

      acc = lax.fori_loop(0, f, _k, jnp.zeros((16,), jnp.float32))
      plsc.store_scatter(wsum_v, [t16, zero16], acc)
      return _

    lax.fori_loop(0, spw // 16, _one, 0)
    pltpu.sync_copy(rows_v, emb_out.at[pl.ds(base, per_w)])
    pltpu.sync_copy(wsum_v,
                    wsum_out.at[pl.ds(wid * spw, spw), pl.ds(0, 1)])

  return gather


def _mlp_body(x_ref, wv_ref, w0_ref, b0_ref, w1_ref, b1_ref, w2_ref, b2_ref,
              wo_ref, bo_ref, o_ref, *, bb):
  xr = x_ref[...].reshape(bb // 8, _G, 8, 128)
  acc = b0_ref[...].astype(jnp.float32)
  h = jnp.zeros((bb, 512), jnp.float32) + acc
  for j in range(_G):
    xj = xr[:, j].reshape(bb, 128).astype(jnp.bfloat16)
    h = h + jnp.dot(xj, w0_ref[j * 128:(j + 1) * 128, :].astype(jnp.bfloat16),
                    preferred_element_type=jnp.float32)
  h = jnp.maximum(h, 0.0)
  h = jnp.maximum(
      jnp.dot(h.astype(jnp.bfloat16), w1_ref[...].astype(jnp.bfloat16),
              preferred_element_type=jnp.float32) + b1_ref[...], 0.0)
  h = jnp.maximum(
      jnp.dot(h.astype(jnp.bfloat16), w2_ref[...].astype(jnp.bfloat16),
              preferred_element_type=jnp.float32) + b2_ref[...], 0.0)
  deep = jnp.dot(h, wo_ref[...], preferred_element_type=jnp.float32)
  wide = wv_ref[:, 0:1]
  logits = deep + wide + bo_ref[0, 0]
  o_ref[...] = 1.0 / (1.0 + jnp.exp(-logits))


def _mlp_call(x, wv, W0p, b0, W1, b1, W2, b2, Wo, bo, bb):
  bsz = wv.shape[0]
  h = W1.shape[0]
  grid = (bsz // bb,)
  return pl.pallas_call(
      functools.partial(_mlp_body, bb=bb),
      grid=grid,
      in_specs=[
          pl.BlockSpec((bb * _G, 128), lambda i: (i, 0)),
          pl.BlockSpec((bb, 128), lambda i: (i, 0)),
          pl.BlockSpec((_G * 128, h), lambda i: (0, 0)),
          pl.BlockSpec((1, h), lambda i: (0, 0)),
          pl.BlockSpec((h, h), lambda i: (0, 0)),
          pl.BlockSpec((1, h), lambda i: (0, 0)),
          pl.BlockSpec((h, h), lambda i: (0, 0)),
          pl.BlockSpec((1, h), lambda i: (0, 0)),
          pl.BlockSpec((h, 1), lambda i: (0, 0)),
          pl.BlockSpec((1, 1), lambda i: (0, 0)),
      ],
      out_specs=pl.BlockSpec((bb, 1), lambda i: (i, 0)),
      out_shape=jax.ShapeDtypeStruct((bsz, 1), jnp.float32),
  )(x, wv, W0p, b0, W1, b1, W2, b2, Wo, bo)


def kernel(inputs, emb_table, wide_table, W0, b0, W1, b1, W2, b2, Wo, bo):
  bsz, f = inputs.shape
  v, d = emb_table.shape
  h = W0.shape[1]

  idx = inputs.astype(jnp.int32)
  idx_pad = jnp.pad(idx, ((0, 0), (0, _G * 4 - f)))
  idx_perm = (idx_pad.reshape(bsz // 8, 8, _G, 4)
              .transpose(0, 2, 1, 3).reshape(bsz * _G * 4))
  wide_flat = wide_table.reshape(v)
  W0p = jnp.pad(W0, ((0, _G * 128 - f * d), (0, 0)))
  b0r, b1r, b2r = b0.reshape(1, h), b1.reshape(1, h), b2.reshape(1, h)
  bor = bo.reshape(1, 1)

  gather = _make_gather(bsz, f, d)
  emb_g, wsum = gather(idx_perm, emb_table, wide_flat)
  x = emb_g.reshape(bsz * _G, 128)
  return _mlp_call(x, wsum, W0p, b0r, W1, b1r, W2, b2r, Wo, bor, bb=256)


# speedup vs baseline: 1.9227x; 1.9227x over previous
import functools

import jax
import jax.numpy as jnp
from jax import lax
from jax.experimental import pallas as pl
from jax.experimental.pallas import tpu as pltpu
from jax.experimental.pallas import tpu_sc as plsc

_NC = 2
_NS = 16
_NW = _NC * _NS
_G = 7


def _make_gather(b, f, d):
  spw = b // _NW
  nin = spw * f
  nq = spw * _G
  npr = nq * 4
  mesh = plsc.VectorSubcoreMesh(core_axis_name="c", subcore_axis_name="s")

  @functools.partial(
      pl.kernel,
      out_type=(
          jax.ShapeDtypeStruct((b * _G, 128), jnp.float32),
          jax.ShapeDtypeStruct((b, 128), jnp.float32),
      ),
      mesh=mesh,
      compiler_params=pltpu.CompilerParams(use_tc_tiling_on_sc=False,
                                           needs_layout_passes=False),
      scratch_types=[
          pltpu.VMEM((nin,), jnp.int32),
          pltpu.VMEM((npr,), jnp.int32),
          pltpu.VMEM((npr, d), jnp.float32),
          pltpu.VMEM((nin,), jnp.float32),
          pltpu.VMEM((spw, 1), jnp.float32),
          pltpu.SemaphoreType.DMA,
          pltpu.SemaphoreType.DMA,
      ],
  )
  def gather(idx_hbm, emb_hbm, wide_hbm, emb_out, wsum_out,
             idx_v, idxp_v, rows_v, wvals_v, wsum_v, sem, wsem):
    wid = lax.axis_index("s") * _NC + lax.axis_index("c")
    pltpu.sync_copy(idx_hbm.at[pl.ds(wid * nin, nin)], idx_v)
    wp = pltpu.async_copy(wide_hbm.at[idx_v], wvals_v, wsem)

    lane = lax.iota(jnp.int32, 16)
    l_hi = lane >> 2
    l_lo = lane & 3
    pad_ok = l_lo < (f - 4 * (_G - 1))

    def _perm(i, _):
      for half in range(2):
        sp = half * 4 + l_hi
        for j in range(_G):
          src = (8 * f) * i + 4 * j + f * sp + l_lo
          if j == _G - 1:
            src = jnp.where(pad_ok, src, 0)
          vals = plsc.load_gather(idx_v, [src])
          dst = nq * l_lo + (8 * _G) * i + 8 * j + sp
          plsc.store_scatter(idxp_v, [dst], vals)
      return _

    lax.fori_loop(0, spw // 8, _perm, 0)
    cp = pltpu.async_copy(emb_hbm.at[idxp_v], rows_v, sem)
    wp.wait()

    zero16 = jnp.zeros((16,), jnp.int32)

    def _one(g, _):
      s16 = g * 16 + lane
      b26 = s16 * f

      def _k(k, acc):
        return acc + plsc.load_gather(wvals_v, [b26 + k])

      acc = lax.fori_loop(0, f, _k, jnp.zeros((16,), jnp.float32))
      plsc.store_scatter(wsum_v, [s16, zero16], acc)
      return _

    lax.fori_loop(0, spw // 16, _one, 0)
    cp.wait()

    for q in range(4):
      pltpu.sync_copy(
          rows_v.at[pl.ds(q * nq, nq)],
          emb_out.at[pl.ds(wid * nq, nq), pl.ds(q * d, d)])
    pltpu.sync_copy(wsum_v,
                    wsum_out.at[pl.ds(wid * spw, spw), pl.ds(0, 1)])

  return gather


def _mlp_body(x_ref, wv_ref, w0_ref, b0_ref, w1_ref, b1_ref, w2_ref, b2_ref,
              wo_ref, bo_ref, o_ref, *, bb):
  xr = x_ref[...].reshape(bb // 8, _G, 8, 128)
  acc = b0_ref[...].astype(jnp.float32)
  h = jnp.zeros((bb, 512), jnp.float32) + acc
  for j in range(_G):
    xj = xr[:, j].reshape(bb, 128).astype(jnp.bfloat16)
    h = h + jnp.dot(xj, w0_ref[j * 128:(j + 1) * 128, :].astype(jnp.bfloat16),
                    preferred_element_type=jnp.float32)
  h = jnp.maximum(h, 0.0)
  h = jnp.maximum(
      jnp.dot(h.astype(jnp.bfloat16), w1_ref[...].astype(jnp.bfloat16),
              preferred_element_type=jnp.float32) + b1_ref[...], 0.0)
  h = jnp.maximum(
      jnp.dot(h.astype(jnp.bfloat16), w2_ref[...].astype(jnp.bfloat16),
              preferred_element_type=jnp.float32) + b2_ref[...], 0.0)
  deep = jnp.dot(h, wo_ref[...], preferred_element_type=jnp.float32)
  wide = wv_ref[:, 0:1]
  logits = deep + wide + bo_ref[0, 0]
  o_ref[...] = 1.0 / (1.0 + jnp.exp(-logits))


def _mlp_call(x, wv, W0p, b0, W1, b1, W2, b2, Wo, bo, bb):
  bsz = wv.shape[0]
  h = W1.shape[0]
  grid = (bsz // bb,)
  return pl.pallas_call(
      functools.partial(_mlp_body, bb=bb),
      grid=grid,
      in_specs=[
          pl.BlockSpec((bb * _G, 128), lambda i: (i, 0)),
          pl.BlockSpec((bb, 128), lambda i: (i, 0)),
          pl.BlockSpec((_G * 128, h), lambda i: (0, 0)),
          pl.BlockSpec((1, h), lambda i: (0, 0)),
          pl.BlockSpec((h, h), lambda i: (0, 0)),
          pl.BlockSpec((1, h), lambda i: (0, 0)),
          pl.BlockSpec((h, h), lambda i: (0, 0)),
          pl.BlockSpec((1, h), lambda i: (0, 0)),
          pl.BlockSpec((h, 1), lambda i: (0, 0)),
          pl.BlockSpec((1, 1), lambda i: (0, 0)),
      ],
      out_specs=pl.BlockSpec((bb, 1), lambda i: (i, 0)),
      out_shape=jax.ShapeDtypeStruct((bsz, 1), jnp.float32),
  )(x, wv, W0p, b0, W1, b1, W2, b2, Wo, bo)


def kernel(inputs, emb_table, wide_table, W0, b0, W1, b1, W2, b2, Wo, bo):
  bsz, f = inputs.shape
  v, d = emb_table.shape
  h = W0.shape[1]

  idx_flat = (inputs.astype(jnp.int32) ^ 0).reshape(bsz * f)
  wide_flat = wide_table.reshape(v)
  W0p = jnp.pad(W0, ((0, _G * 128 - f * d), (0, 0)))
  b0r, b1r, b2r = b0.reshape(1, h), b1.reshape(1, h), b2.reshape(1, h)
  bor = bo.reshape(1, 1)

  gather = _make_gather(bsz, f, d)
  emb_g, wsum = gather(idx_flat, emb_table, wide_flat)
  return _mlp_call(emb_g, wsum, W0p, b0r, W1, b1r, W2, b2r, Wo, bor, bb=256)


# speedup vs baseline: 1.9644x; 1.0217x over previous
import functools

import jax
import jax.numpy as jnp
from jax import lax
from jax.experimental import pallas as pl
from jax.experimental.pallas import tpu as pltpu
from jax.experimental.pallas import tpu_sc as plsc

_NC = 2
_NS = 16
_NW = _NC * _NS
_G = 7


def _make_gather(b, f, d):
  spw = b // _NW
  nin = spw * f
  nq = spw * _G
  npr = nq * 4
  mesh = plsc.VectorSubcoreMesh(core_axis_name="c", subcore_axis_name="s")

  @functools.partial(
      pl.kernel,
      out_type=(
          jax.ShapeDtypeStruct((b * _G, 128), jnp.float32),
          jax.ShapeDtypeStruct((b, 128), jnp.float32),
      ),
      mesh=mesh,
      compiler_params=pltpu.CompilerParams(use_tc_tiling_on_sc=False,
                                           needs_layout_passes=False),
      scratch_types=[
          pltpu.VMEM((spw, 32), jnp.int32),
          pltpu.VMEM((npr,), jnp.int32),
          pltpu.VMEM((npr, d), jnp.float32),
          pltpu.VMEM((npr,), jnp.float32),
          pltpu.VMEM((spw, 1), jnp.float32),
          pltpu.SemaphoreType.DMA,
          pltpu.SemaphoreType.DMA,
      ],
  )
  def gather(idx_hbm, emb_hbm, wide_hbm, emb_out, wsum_out,
             idx_v, idxp_v, rows_v, wvals_v, wsum_v, sem, wsem):
    wid = lax.axis_index("s") * _NC + lax.axis_index("c")
    pltpu.sync_copy(idx_hbm.at[pl.ds(wid * spw, spw), pl.ds(0, 32)], idx_v)

    lane = lax.iota(jnp.int32, 16)
    l_hi = lane >> 2
    l_lo = lane & 3
    pad_ok = l_lo < (f - 4 * (_G - 1))

    def _perm(i, _):
      for half in range(2):
        sp = half * 4 + l_hi
        for j in range(_G):
          cols = jnp.full((16,), 4 * j, jnp.int32) + l_lo
          if j == _G - 1:
            cols = jnp.where(pad_ok, cols, 0)
          vals = plsc.load_gather(idx_v, [8 * i + sp, cols])
          dst = nq * l_lo + (8 * _G) * i + 8 * j + sp
          plsc.store_scatter(idxp_v, [dst], vals)
      return _

    lax.fori_loop(0, spw // 8, _perm, 0)
    cp = pltpu.async_copy(emb_hbm.at[idxp_v], rows_v, sem)
    wp = pltpu.async_copy(wide_hbm.at[idxp_v], wvals_v, wsem)
    wp.wait()

    zero16 = jnp.zeros((16,), jnp.int32)

    def _one(g, _):
      s16 = g * 16 + lane
      boff = (8 * _G) * (s16 >> 3) + (s16 & 7)

      def _k(k, acc):
        off = boff + nq * (k & 3) + 8 * (k >> 2)
        return acc + plsc.load_gather(wvals_v, [off])

      acc = lax.fori_loop(0, f, _k, jnp.zeros((16,), jnp.float32))
      plsc.store_scatter(wsum_v, [s16, zero16], acc)
      return _

    lax.fori_loop(0, spw // 16, _one, 0)
    cp.wait()

    for q in range(4):
      pltpu.sync_copy(
          rows_v.at[pl.ds(q * nq, nq)],
          emb_out.at[pl.ds(wid * nq, nq), pl.ds(q * d, d)])
    pltpu.sync_copy(wsum_v,
                    wsum_out.at[pl.ds(wid * spw, spw), pl.ds(0, 1)])

  return gather


def _mlp_body(x_ref, wv_ref, w0_ref, b0_ref, w1_ref, b1_ref, w2_ref, b2_ref,
              wo_ref, bo_ref, o_ref, *, bb):
  xr = x_ref[...].reshape(bb // 8, _G, 8, 128)
  acc = b0_ref[...].astype(jnp.float32)
  h = jnp.zeros((bb, 512), jnp.float32) + acc
  for j in range(_G):
    xj = xr[:, j].reshape(bb, 128).astype(jnp.bfloat16)
    h = h + jnp.dot(xj, w0_ref[j * 128:(j + 1) * 128, :].astype(jnp.bfloat16),
                    preferred_element_type=jnp.float32)
  h = jnp.maximum(h, 0.0)
  h = jnp.maximum(
      jnp.dot(h.astype(jnp.bfloat16), w1_ref[...].astype(jnp.bfloat16),
              preferred_element_type=jnp.float32) + b1_ref[...], 0.0)
  h = jnp.maximum(
      jnp.dot(h.astype(jnp.bfloat16), w2_ref[...].astype(jnp.bfloat16),
              preferred_element_type=jnp.float32) + b2_ref[...], 0.0)
  deep = jnp.dot(h, wo_ref[...], preferred_element_type=jnp.float32)
  wide = wv_ref[:, 0:1]
  logits = deep + wide + bo_ref[0, 0]
  o_ref[...] = 1.0 / (1.0 + jnp.exp(-logits))


def _mlp_call(x, wv, W0p, b0, W1, b1, W2, b2, Wo, bo, bb):
  bsz = wv.shape[0]
  h = W1.shape[0]
  grid = (bsz // bb,)
  return pl.pallas_call(
      functools.partial(_mlp_body, bb=bb),
      grid=grid,
      in_specs=[
          pl.BlockSpec((bb * _G, 128), lambda i: (i, 0)),
          pl.BlockSpec((bb, 128), lambda i: (i, 0)),
          pl.BlockSpec((_G * 128, h), lambda i: (0, 0)),
          pl.BlockSpec((1, h), lambda i: (0, 0)),
          pl.BlockSpec((h, h), lambda i: (0, 0)),
          pl.BlockSpec((1, h), lambda i: (0, 0)),
          pl.BlockSpec((h, h), lambda i: (0, 0)),
          pl.BlockSpec((1, h), lambda i: (0, 0)),
          pl.BlockSpec((h, 1), lambda i: (0, 0)),
          pl.BlockSpec((1, 1), lambda i: (0, 0)),
      ],
      out_specs=pl.BlockSpec((bb, 1), lambda i: (i, 0)),
      out_shape=jax.ShapeDtypeStruct((bsz, 1), jnp.float32),
  )(x, wv, W0p, b0, W1, b1, W2, b2, Wo, bo)


def kernel(inputs, emb_table, wide_table, W0, b0, W1, b1, W2, b2, Wo, bo):
  bsz, f = inputs.shape
  v, d = emb_table.shape
  h = W0.shape[1]

  idx_pad = jnp.pad(inputs.astype(jnp.int32) ^ 0, ((0, 0), (0, 128 - f)))
  wide_flat = wide_table.reshape(v)
  W0p = jnp.pad(W0, ((0, _G * 128 - f * d), (0, 0)))
  b0r, b1r, b2r = b0.reshape(1, h), b1.reshape(1, h), b2.reshape(1, h)
  bor = bo.reshape(1, 1)

  gather = _make_gather(bsz, f, d)
  emb_g, wsum = gather(idx_pad, emb_table, wide_flat)
  return _mlp_call(emb_g, wsum, W0p, b0r, W1, b1r, W2, b2r, Wo, bor, bb=256)
